# baseline (speedup 1.0000x reference)
import numpy as np
import jax
import jax.numpy as jnp
from jax.experimental import pallas as pl
from jax.experimental.pallas import tpu as pltpu


def _interp_matrix(out_size, in_size, align_corners=False):
    if in_size == 1:
        return np.ones((out_size, 1), np.float32)
    if align_corners:
        src = np.arange(out_size, dtype=np.float64) * (in_size - 1) / max(out_size - 1, 1)
    else:
        src = (np.arange(out_size, dtype=np.float64) + 0.5) * (in_size / out_size) - 0.5
        src = np.clip(src, 0.0, in_size - 1.0)
    i0 = np.clip(np.floor(src).astype(np.int64), 0, in_size - 2)
    frac = (src - i0).astype(np.float32)
    m = np.zeros((out_size, in_size), np.float32)
    m[np.arange(out_size), i0] += 1.0 - frac
    m[np.arange(out_size), i0 + 1] += frac
    return m


def _fused_kernel(x_ref, w_ref, a_ref, bt_ref, xh_ref, o_ref, yt_ref):
    cout = w_ref.shape[0]
    hl, wl = yt_ref.shape[0], bt_ref.shape[0]
    th, wh = a_ref.shape[0], bt_ref.shape[1]

    @pl.when(pl.program_id(1) == 0)
    def _conv():
        y = jnp.dot(w_ref[...].astype(jnp.float32),
                    x_ref[0].astype(jnp.float32),
                    preferred_element_type=jnp.float32)
        yt = y.reshape(cout, hl, wl).transpose(1, 0, 2)
        yt_ref[...] = yt.reshape(hl, cout * wl)

    zt = jnp.dot(a_ref[...], yt_ref[...],
                 preferred_element_type=jnp.float32)
    zh = zt.reshape(th, cout, wl).transpose(1, 0, 2)
    up = jnp.dot(zh.reshape(cout * th, wl), bt_ref[...],
                 preferred_element_type=jnp.float32)

    o_ref[0, :cout] = up.reshape(cout, th, wh).astype(o_ref.dtype)
    o_ref[0, cout:] = xh_ref[0]


def _pick_row_tile(hh):
    if hh % 8 != 0:
        return hh
    best = 8
    for t in range(8, hh + 1, 8):
        if hh % t == 0 and hh // t >= 2 and t <= 16:
            best = t
    return best


def kernel(input_low, input_high, w):
    n, cin, hl, wl = input_low.shape
    nh, ch, hh, wh = input_high.shape
    cout = w.shape[0]
    ctot = cout + ch
    dtype = input_high.dtype
    isz = jnp.dtype(dtype).itemsize

    a = jnp.asarray(_interp_matrix(hh, hl))
    bt = jnp.asarray(_interp_matrix(wh, wl).T)

    th = _pick_row_tile(hh)
    n_t = hh // th

    blocks = (cin * hl * wl * isz + th * hl * 4 + wl * wh * 4
              + ch * th * wh * isz + ctot * th * wh * isz)
    scratch = 4 * hl * cout * wl + 4 * cout * (hl * wl + th * (hl + wl + 2 * wh))
    vmem_limit = int(min(100 << 20, max(32 << 20, 2 * (2 * blocks + scratch))))

    out = pl.pallas_call(
        _fused_kernel,
        out_shape=jax.ShapeDtypeStruct((n, ctot, hh, wh), dtype),
        grid=(n, n_t),
        in_specs=[
            pl.BlockSpec((1, cin, hl * wl), lambda i, t: (i, 0, 0)),
            pl.BlockSpec((cout, cin), lambda i, t: (0, 0)),
            pl.BlockSpec((th, hl), lambda i, t: (t, 0)),
            pl.BlockSpec((wl, wh), lambda i, t: (0, 0)),
            pl.BlockSpec((1, ch, th, wh), lambda i, t: (i, 0, t, 0)),
        ],
        out_specs=pl.BlockSpec((1, ctot, th, wh), lambda i, t: (i, 0, t, 0)),
        scratch_shapes=[pltpu.VMEM((hl, cout * wl), jnp.float32)],
        compiler_params=pltpu.CompilerParams(
            dimension_semantics=("parallel", "arbitrary"),
            vmem_limit_bytes=vmem_limit),
        cost_estimate=pl.CostEstimate(
            flops=int(2 * n * cout * (cin * hl * wl + hh * hl * wl + hh * wl * wh)),
            transcendentals=0,
            bytes_accessed=int(isz * n * (cin * hl * wl + ch * hh * wh
                                          + ctot * hh * wh)
                               + 4 * (hh * hl + wl * wh))),
    )(input_low.reshape(n, cin, hl * wl), w, a, bt, input_high)
    return out


# speedup vs baseline: 1.2530x; 1.2530x over previous
import numpy as np
import jax
import jax.numpy as jnp
from jax.experimental import pallas as pl
from jax.experimental.pallas import tpu as pltpu


def _interp_matrix(out_size, in_size, align_corners=False):
    if in_size == 1:
        return np.ones((out_size, 1), np.float32)
    if align_corners:
        src = np.arange(out_size, dtype=np.float64) * (in_size - 1) / max(out_size - 1, 1)
    else:
        src = (np.arange(out_size, dtype=np.float64) + 0.5) * (in_size / out_size) - 0.5
        src = np.clip(src, 0.0, in_size - 1.0)
    i0 = np.clip(np.floor(src).astype(np.int64), 0, in_size - 2)
    frac = (src - i0).astype(np.float32)
    m = np.zeros((out_size, in_size), np.float32)
    m[np.arange(out_size), i0] += 1.0 - frac
    m[np.arange(out_size), i0 + 1] += frac
    return m


def _fused_kernel(x_ref, w_ref, a_ref, bt_ref, xh_ref, o_ref, yt_ref):
    cout = w_ref.shape[0]
    hl, wl = yt_ref.shape[0], bt_ref.shape[0]
    th, wh = a_ref.shape[0], bt_ref.shape[1]

    @pl.when(pl.program_id(1) == 0)
    def _conv():
        y = jnp.dot(w_ref[...].astype(jnp.float32),
                    x_ref[0].astype(jnp.float32),
                    preferred_element_type=jnp.float32)
        yt = y.reshape(cout, hl, wl).transpose(1, 0, 2)
        yt_ref[...] = yt.reshape(hl, cout * wl)

    zt = jnp.dot(a_ref[...], yt_ref[...],
                 preferred_element_type=jnp.float32)
    zh = zt.reshape(th, cout, wl).transpose(1, 0, 2)
    up = jnp.dot(zh.reshape(cout * th, wl), bt_ref[...],
                 preferred_element_type=jnp.float32)

    o_ref[0, :cout] = up.reshape(cout, th, wh).astype(o_ref.dtype)
    o_ref[0, cout:] = xh_ref[0]


def _pick_row_tile(hh):
    if hh % 8 != 0:
        return hh
    best = 8
    for t in range(8, hh + 1, 8):
        if hh % t == 0 and hh // t >= 2 and t <= 64:
            best = t
    return best


def kernel(input_low, input_high, w):
    n, cin, hl, wl = input_low.shape
    nh, ch, hh, wh = input_high.shape
    cout = w.shape[0]
    ctot = cout + ch
    dtype = input_high.dtype
    isz = jnp.dtype(dtype).itemsize

    a = jnp.asarray(_interp_matrix(hh, hl))
    bt = jnp.asarray(_interp_matrix(wh, wl).T)

    th = _pick_row_tile(hh)
    n_t = hh // th

    blocks = (cin * hl * wl * isz + th * hl * 4 + wl * wh * 4
              + ch * th * wh * isz + ctot * th * wh * isz)
    scratch = 4 * hl * cout * wl + 4 * cout * (hl * wl + th * (hl + wl + 2 * wh))
    vmem_limit = int(min(100 << 20, max(32 << 20, 2 * (2 * blocks + scratch))))

    out = pl.pallas_call(
        _fused_kernel,
        out_shape=jax.ShapeDtypeStruct((n, ctot, hh, wh), dtype),
        grid=(n, n_t),
        in_specs=[
            pl.BlockSpec((1, cin, hl * wl), lambda i, t: (i, 0, 0)),
            pl.BlockSpec((cout, cin), lambda i, t: (0, 0)),
            pl.BlockSpec((th, hl), lambda i, t: (t, 0)),
            pl.BlockSpec((wl, wh), lambda i, t: (0, 0)),
            pl.BlockSpec((1, ch, th, wh), lambda i, t: (i, 0, t, 0)),
        ],
        out_specs=pl.BlockSpec((1, ctot, th, wh), lambda i, t: (i, 0, t, 0)),
        scratch_shapes=[pltpu.VMEM((hl, cout * wl), jnp.float32)],
        compiler_params=pltpu.CompilerParams(
            dimension_semantics=("parallel", "arbitrary"),
            vmem_limit_bytes=vmem_limit),
        cost_estimate=pl.CostEstimate(
            flops=int(2 * n * cout * (cin * hl * wl + hh * hl * wl + hh * wl * wh)),
            transcendentals=0,
            bytes_accessed=int(isz * n * (cin * hl * wl + ch * hh * wh
                                          + ctot * hh * wh)
                               + 4 * (hh * hl + wl * wh))),
    )(input_low.reshape(n, cin, hl * wl), w, a, bt, input_high)
    return out
